# baseline (speedup 1.0000x reference)
import functools

import jax
import jax.numpy as jnp
from jax import lax
from jax.experimental import pallas as pl
from jax.experimental.pallas import tpu as pltpu
from jax.experimental.pallas import tpu_sc as plsc

N = 160000
D = 256
H = 256
M = 10000
OUT = 128
DH = 128

NC = 2
NS = 16
ROWS_PER_TILE = N // NS
CH = 40
NCH = ROWS_PER_TILE // CH
SEG_BLK = 1000
SEG_TILES = M // SEG_BLK

PHI_BLK = 1000
RHO_BLK = 1000


def _phi_body(x_ref, w1_ref, b1_ref, w2_ref, b2_ref, out_ref):
    x = x_ref[...].astype(jnp.bfloat16)
    h = jnp.maximum(
        jnp.dot(x, w1_ref[...], preferred_element_type=jnp.float32)
        + b1_ref[...], 0.0)
    y = jnp.dot(h.astype(jnp.bfloat16), w2_ref[...],
                preferred_element_type=jnp.float32) + b2_ref[...]
    out_ref[0] = y[:, :DH]
    out_ref[1] = y[:, DH:]


def _phi_halves(x, w1, b1, w2, b2):
    grid = (N // PHI_BLK,)
    return pl.pallas_call(
        _phi_body,
        grid=grid,
        in_specs=[
            pl.BlockSpec((PHI_BLK, D), lambda i: (i, 0)),
            pl.BlockSpec((D, H), lambda i: (0, 0)),
            pl.BlockSpec((1, H), lambda i: (0, 0)),
            pl.BlockSpec((H, D), lambda i: (0, 0)),
            pl.BlockSpec((1, D), lambda i: (0, 0)),
        ],
        out_specs=pl.BlockSpec((NC, PHI_BLK, DH), lambda i: (0, i, 0)),
        out_shape=jax.ShapeDtypeStruct((NC, N, DH), jnp.float32),
        compiler_params=pltpu.CompilerParams(
            dimension_semantics=("arbitrary",)),
    )(x, w1.astype(jnp.bfloat16), b1.reshape(1, H),
      w2.astype(jnp.bfloat16), b2.reshape(1, D))


NBUF = 5
RDEPTH = 2


def _seg_body(phi_ref, ids_ref, zeros_ref, out_ref,
              acc, buf, ibuf, *sems):
    c = lax.axis_index("c")
    s = lax.axis_index("s")
    row0 = s * ROWS_PER_TILE
    srd = sems[:NBUF]
    ssc = sems[NBUF:]

    def rd_descs(k, b):
        base = row0 + k * CH
        dr = pltpu.make_async_copy(
            phi_ref.at[c, pl.ds(base, CH)], buf.at[b], srd[b])
        di = pltpu.make_async_copy(
            ids_ref.at[pl.ds(base, CH)], ibuf.at[b], srd[b])
        return dr, di

    def sc_desc(b):
        return pltpu.async_copy(buf.at[b], acc.at[ibuf.at[b]], ssc[b],
                                add=True)

    def sc_wait(b):
        pltpu.make_async_copy(buf.at[b], acc.at[ibuf.at[b]], ssc[b]).wait()

    @pl.when(s < SEG_TILES)
    def _():
        pltpu.sync_copy(zeros_ref, acc.at[pl.ds(s * SEG_BLK, SEG_BLK)])
    plsc.subcore_barrier()

    for b in range(RDEPTH):
        dr, di = rd_descs(b, b)
        dr.start()
        di.start()

    def step(k, b):
        dr, di = rd_descs(k, b)
        dr.wait()
        di.wait()
        sc_desc(b)
        j = k - (NBUF - RDEPTH)
        bj = (b + RDEPTH) % NBUF

        @pl.when(j >= 0)
        def _():
            sc_wait(bj)

        @pl.when(k + RDEPTH < NCH)
        def _():
            dr2, di2 = rd_descs(k + RDEPTH, bj)
            dr2.start()
            di2.start()

    @pl.loop(0, NCH, step=NBUF)
    def _(k0):
        for i in range(NBUF):
            step(k0 + i, i)

    for k in range(NCH - (NBUF - RDEPTH), NCH):
        sc_wait(k % NBUF)

    plsc.subcore_barrier()

    @pl.when(s < SEG_TILES)
    def _():
        pltpu.sync_copy(acc.at[pl.ds(s * SEG_BLK, SEG_BLK)],
                        out_ref.at[c, pl.ds(s * SEG_BLK, SEG_BLK)])


def _segment_sum(phi_halves, ids):
    zeros = jnp.zeros((SEG_BLK, DH), jnp.float32)
    fn = pl.kernel(
        _seg_body,
        out_type=jax.ShapeDtypeStruct((NC, M, DH), jnp.float32),
        mesh=plsc.VectorSubcoreMesh(
            core_axis_name="c", subcore_axis_name="s",
            num_cores=NC, num_subcores=NS),
        scratch_types=[
            pltpu.VMEM_SHARED((M, DH), jnp.float32),
            pltpu.VMEM((NBUF, CH, DH), jnp.float32),
            pltpu.VMEM((NBUF, CH), jnp.int32),
        ] + [pltpu.SemaphoreType.DMA] * (2 * NBUF),
    )
    return fn(phi_halves, ids, zeros)


def _rho_body(x_ref, w1_ref, b1_ref, w2_ref, b2_ref, out_ref):
    g = jnp.maximum(
        jnp.dot(x_ref[0], w1_ref[:DH, :], preferred_element_type=jnp.float32)
        + jnp.dot(x_ref[1], w1_ref[DH:, :], preferred_element_type=jnp.float32)
        + b1_ref[...], 0.0)
    out_ref[...] = jnp.dot(
        g, w2_ref[...], preferred_element_type=jnp.float32) + b2_ref[...]


def _rho(phi_mols, w1, b1, w2, b2):
    grid = (M // RHO_BLK,)
    return pl.pallas_call(
        _rho_body,
        grid=grid,
        in_specs=[
            pl.BlockSpec((NC, RHO_BLK, DH), lambda i: (0, i, 0)),
            pl.BlockSpec((D, H), lambda i: (0, 0)),
            pl.BlockSpec((1, H), lambda i: (0, 0)),
            pl.BlockSpec((H, OUT), lambda i: (0, 0)),
            pl.BlockSpec((1, OUT), lambda i: (0, 0)),
        ],
        out_specs=pl.BlockSpec((RHO_BLK, OUT), lambda i: (i, 0)),
        out_shape=jax.ShapeDtypeStruct((M, OUT), jnp.float32),
        compiler_params=pltpu.CompilerParams(
            dimension_semantics=("arbitrary",)),
    )(phi_mols, w1, b1.reshape(1, H), w2, b2.reshape(1, OUT))


@jax.jit
def kernel(f_subgraphs, phi_w1, phi_b1, phi_w2, phi_b2,
           rho_w1, rho_b1, rho_w2, rho_b2, segment_ids):
    phi_halves = _phi_halves(f_subgraphs, phi_w1, phi_b1, phi_w2, phi_b2)
    ids = segment_ids.astype(jnp.int32)
    phi_mols = _segment_sum(phi_halves, ids)
    return _rho(phi_mols, rho_w1, rho_b1, rho_w2, rho_b2)


# speedup vs baseline: 1.2658x; 1.2658x over previous
import functools

import jax
import jax.numpy as jnp
from jax import lax
from jax.experimental import pallas as pl
from jax.experimental.pallas import tpu as pltpu
from jax.experimental.pallas import tpu_sc as plsc

N = 160000
D = 256
H = 256
M = 10000
OUT = 128
DH = 128

NC = 2
NS = 16
ROWS_PER_TILE = N // NS
CH = 80
NCH = ROWS_PER_TILE // CH
SEG_BLK = 2000
SEG_TILES = M // SEG_BLK

PHI_BLK = 2000
RHO_BLK = 1000


def _phi_body(x_ref, w1_ref, b1_ref, w2_ref, b2_ref, out_ref):
    x = x_ref[...].astype(jnp.bfloat16)
    h = jnp.maximum(
        jnp.dot(x, w1_ref[...], preferred_element_type=jnp.float32)
        + b1_ref[...], 0.0)
    y = jnp.dot(h.astype(jnp.bfloat16), w2_ref[...],
                preferred_element_type=jnp.float32) + b2_ref[...]
    out_ref[0] = y[:, :DH]
    out_ref[1] = y[:, DH:]


def _phi_halves(x, w1, b1, w2, b2):
    grid = (N // PHI_BLK,)
    return pl.pallas_call(
        _phi_body,
        grid=grid,
        in_specs=[
            pl.BlockSpec((PHI_BLK, D), lambda i: (i, 0)),
            pl.BlockSpec((D, H), lambda i: (0, 0)),
            pl.BlockSpec((1, H), lambda i: (0, 0)),
            pl.BlockSpec((H, D), lambda i: (0, 0)),
            pl.BlockSpec((1, D), lambda i: (0, 0)),
        ],
        out_specs=pl.BlockSpec((NC, PHI_BLK, DH), lambda i: (0, i, 0)),
        out_shape=jax.ShapeDtypeStruct((NC, N, DH), jnp.float32),
        compiler_params=pltpu.CompilerParams(
            dimension_semantics=("arbitrary",)),
    )(x, w1.astype(jnp.bfloat16), b1.reshape(1, H),
      w2.astype(jnp.bfloat16), b2.reshape(1, D))


NBUF = 4
RDEPTH = 2


def _seg_body(phi_ref, ids_ref, zeros_ref, out_ref,
              acc, buf, ibuf, *sems):
    c = lax.axis_index("c")
    s = lax.axis_index("s")
    row0 = s * ROWS_PER_TILE
    srd = sems[:NBUF]
    ssc = sems[NBUF:]

    def rd_descs(k, b):
        base = row0 + k * CH
        dr = pltpu.make_async_copy(
            phi_ref.at[c, pl.ds(base, CH)], buf.at[b], srd[b])
        di = pltpu.make_async_copy(
            ids_ref.at[pl.ds(base, CH)], ibuf.at[b], srd[b])
        return dr, di

    def sc_desc(b):
        return pltpu.async_copy(buf.at[b], acc.at[ibuf.at[b]], ssc[b],
                                add=True)

    def sc_wait(b):
        pltpu.make_async_copy(buf.at[b], acc.at[ibuf.at[b]], ssc[b]).wait()

    @pl.when(s < SEG_TILES)
    def _():
        pltpu.sync_copy(zeros_ref, acc.at[pl.ds(s * SEG_BLK, SEG_BLK)])
    plsc.subcore_barrier()

    for b in range(RDEPTH):
        dr, di = rd_descs(b, b)
        dr.start()
        di.start()

    def step(k, b):
        dr, di = rd_descs(k, b)
        dr.wait()
        di.wait()
        sc_desc(b)
        j = k - (NBUF - RDEPTH)
        bj = (b + RDEPTH) % NBUF

        @pl.when(j >= 0)
        def _():
            sc_wait(bj)

        @pl.when(k + RDEPTH < NCH)
        def _():
            dr2, di2 = rd_descs(k + RDEPTH, bj)
            dr2.start()
            di2.start()

    main = (NCH // NBUF) * NBUF

    @pl.loop(0, main, step=NBUF)
    def _(k0):
        for i in range(NBUF):
            step(k0 + i, i)

    for k in range(main, NCH):
        step(k, k % NBUF)
    for k in range(NCH - (NBUF - RDEPTH), NCH):
        sc_wait(k % NBUF)

    plsc.subcore_barrier()

    @pl.when(s < SEG_TILES)
    def _():
        pltpu.sync_copy(acc.at[pl.ds(s * SEG_BLK, SEG_BLK)],
                        out_ref.at[c, pl.ds(s * SEG_BLK, SEG_BLK)])


def _segment_sum(phi_halves, ids):
    zeros = jnp.zeros((SEG_BLK, DH), jnp.float32)
    fn = pl.kernel(
        _seg_body,
        out_type=jax.ShapeDtypeStruct((NC, M, DH), jnp.float32),
        mesh=plsc.VectorSubcoreMesh(
            core_axis_name="c", subcore_axis_name="s",
            num_cores=NC, num_subcores=NS),
        scratch_types=[
            pltpu.VMEM_SHARED((M, DH), jnp.float32),
            pltpu.VMEM((NBUF, CH, DH), jnp.float32),
            pltpu.VMEM((NBUF, CH), jnp.int32),
        ] + [pltpu.SemaphoreType.DMA] * (2 * NBUF),
    )
    return fn(phi_halves, ids, zeros)


def _rho_body(x_ref, w1_ref, b1_ref, w2_ref, b2_ref, out_ref):
    g = jnp.maximum(
        jnp.dot(x_ref[0], w1_ref[:DH, :], preferred_element_type=jnp.float32)
        + jnp.dot(x_ref[1], w1_ref[DH:, :], preferred_element_type=jnp.float32)
        + b1_ref[...], 0.0)
    out_ref[...] = jnp.dot(
        g, w2_ref[...], preferred_element_type=jnp.float32) + b2_ref[...]


def _rho(phi_mols, w1, b1, w2, b2):
    grid = (M // RHO_BLK,)
    return pl.pallas_call(
        _rho_body,
        grid=grid,
        in_specs=[
            pl.BlockSpec((NC, RHO_BLK, DH), lambda i: (0, i, 0)),
            pl.BlockSpec((D, H), lambda i: (0, 0)),
            pl.BlockSpec((1, H), lambda i: (0, 0)),
            pl.BlockSpec((H, OUT), lambda i: (0, 0)),
            pl.BlockSpec((1, OUT), lambda i: (0, 0)),
        ],
        out_specs=pl.BlockSpec((RHO_BLK, OUT), lambda i: (i, 0)),
        out_shape=jax.ShapeDtypeStruct((M, OUT), jnp.float32),
        compiler_params=pltpu.CompilerParams(
            dimension_semantics=("arbitrary",)),
    )(phi_mols, w1, b1.reshape(1, H), w2, b2.reshape(1, OUT))


@jax.jit
def kernel(f_subgraphs, phi_w1, phi_b1, phi_w2, phi_b2,
           rho_w1, rho_b1, rho_w2, rho_b2, segment_ids):
    phi_halves = _phi_halves(f_subgraphs, phi_w1, phi_b1, phi_w2, phi_b2)
    ids = segment_ids.astype(jnp.int32)
    phi_mols = _segment_sum(phi_halves, ids)
    return _rho(phi_mols, rho_w1, rho_b1, rho_w2, rho_b2)


# speedup vs baseline: 1.3546x; 1.0701x over previous
import functools

import jax
import jax.numpy as jnp
from jax import lax
from jax.experimental import pallas as pl
from jax.experimental.pallas import tpu as pltpu
from jax.experimental.pallas import tpu_sc as plsc

N = 160000
D = 256
H = 256
M = 10000
OUT = 128
DH = 128

NC = 2
NS = 16
CH = 80
SEG_BLK = 2000
SEG_TILES = M // SEG_BLK

PHI_BLK = 3200
RHO_BLK = 1000

STAGE_SIZES = (57600, 51200, 51200)
assert sum(STAGE_SIZES) == N


def _phi_body(x_ref, w1_ref, b1_ref, w2_ref, b2_ref, out_ref):
    x = x_ref[...].astype(jnp.bfloat16)
    h = jnp.maximum(
        jnp.dot(x, w1_ref[...], preferred_element_type=jnp.float32)
        + b1_ref[...], 0.0)
    y = jnp.dot(h.astype(jnp.bfloat16), w2_ref[...],
                preferred_element_type=jnp.float32) + b2_ref[...]
    out_ref[0] = y[:, :DH]
    out_ref[1] = y[:, DH:]


def _phi_stage(x, w1, b1, w2, b2, row0, count):
    blk0 = row0 // PHI_BLK
    return pl.pallas_call(
        _phi_body,
        grid=(count // PHI_BLK,),
        in_specs=[
            pl.BlockSpec((PHI_BLK, D), lambda i: (blk0 + i, 0)),
            pl.BlockSpec((D, H), lambda i: (0, 0)),
            pl.BlockSpec((1, H), lambda i: (0, 0)),
            pl.BlockSpec((H, D), lambda i: (0, 0)),
            pl.BlockSpec((1, D), lambda i: (0, 0)),
        ],
        out_specs=pl.BlockSpec((NC, PHI_BLK, DH), lambda i: (0, i, 0)),
        out_shape=jax.ShapeDtypeStruct((NC, count, DH), jnp.float32),
        compiler_params=pltpu.CompilerParams(
            dimension_semantics=("arbitrary",)),
    )(x, w1, b1, w2, b2)


NBUF = 4
RDEPTH = 2


def _make_seg_body(ids_row0, rows_per_tile):
    nch = rows_per_tile // CH

    def _seg_body(phi_ref, ids_ref, zeros_ref, out_ref,
                  acc, buf, ibuf, *sems):
        c = lax.axis_index("c")
        s = lax.axis_index("s")
        row0 = s * rows_per_tile
        srd = sems[:NBUF]
        ssc = sems[NBUF:]

        def rd_descs(k, b):
            base = row0 + k * CH
            dr = pltpu.make_async_copy(
                phi_ref.at[c, pl.ds(base, CH)], buf.at[b], srd[b])
            di = pltpu.make_async_copy(
                ids_ref.at[pl.ds(ids_row0 + base, CH)], ibuf.at[b], srd[b])
            return dr, di

        def sc_start(b):
            pltpu.async_copy(buf.at[b], acc.at[ibuf.at[b]], ssc[b], add=True)

        def sc_wait(b):
            pltpu.make_async_copy(
                buf.at[b], acc.at[ibuf.at[b]], ssc[b]).wait()

        @pl.when(s < SEG_TILES)
        def _():
            pltpu.sync_copy(zeros_ref, acc.at[pl.ds(s * SEG_BLK, SEG_BLK)])
        plsc.subcore_barrier()

        for b in range(RDEPTH):
            dr, di = rd_descs(b, b)
            dr.start()
            di.start()

        def step(k, b):
            dr, di = rd_descs(k, b)
            dr.wait()
            di.wait()
            sc_start(b)
            bj = (b + RDEPTH) % NBUF

            @pl.when(k - (NBUF - RDEPTH) >= 0)
            def _():
                sc_wait(bj)

            @pl.when(k + RDEPTH < nch)
            def _():
                dr2, di2 = rd_descs(k + RDEPTH, bj)
                dr2.start()
                di2.start()

        main = (nch // NBUF) * NBUF

        @pl.loop(0, main, step=NBUF)
        def _(k0):
            for i in range(NBUF):
                step(k0 + i, i)

        for k in range(main, nch):
            step(k, k % NBUF)
        for k in range(nch - (NBUF - RDEPTH), nch):
            sc_wait(k % NBUF)

        plsc.subcore_barrier()

        @pl.when(s < SEG_TILES)
        def _():
            pltpu.sync_copy(acc.at[pl.ds(s * SEG_BLK, SEG_BLK)],
                            out_ref.at[c, pl.ds(s * SEG_BLK, SEG_BLK)])

    return _seg_body


def _segment_sum_stage(phi_halves, ids, zeros, row0, count):
    fn = pl.kernel(
        _make_seg_body(row0, count // NS),
        out_type=jax.ShapeDtypeStruct((NC, M, DH), jnp.float32),
        mesh=plsc.VectorSubcoreMesh(
            core_axis_name="c", subcore_axis_name="s",
            num_cores=NC, num_subcores=NS),
        scratch_types=[
            pltpu.VMEM_SHARED((M, DH), jnp.float32),
            pltpu.VMEM((NBUF, CH, DH), jnp.float32),
            pltpu.VMEM((NBUF, CH), jnp.int32),
        ] + [pltpu.SemaphoreType.DMA] * (2 * NBUF),
    )
    return fn(phi_halves, ids, zeros)


def _rho_body(*refs):
    nstage = len(STAGE_SIZES)
    parts = refs[:nstage]
    w1_ref, b1_ref, w2_ref, b2_ref, out_ref = refs[nstage:]
    xlo = parts[0][0]
    xhi = parts[0][1]
    for p in parts[1:]:
        xlo = xlo + p[0]
        xhi = xhi + p[1]
    g = jnp.maximum(
        jnp.dot(xlo, w1_ref[:DH, :], preferred_element_type=jnp.float32)
        + jnp.dot(xhi, w1_ref[DH:, :], preferred_element_type=jnp.float32)
        + b1_ref[...], 0.0)
    out_ref[...] = jnp.dot(
        g, w2_ref[...], preferred_element_type=jnp.float32) + b2_ref[...]


def _rho(partials, w1, b1, w2, b2):
    grid = (M // RHO_BLK,)
    part_spec = pl.BlockSpec((NC, RHO_BLK, DH), lambda i: (0, i, 0))
    return pl.pallas_call(
        _rho_body,
        grid=grid,
        in_specs=[part_spec] * len(partials) + [
            pl.BlockSpec((D, H), lambda i: (0, 0)),
            pl.BlockSpec((1, H), lambda i: (0, 0)),
            pl.BlockSpec((H, OUT), lambda i: (0, 0)),
            pl.BlockSpec((1, OUT), lambda i: (0, 0)),
        ],
        out_specs=pl.BlockSpec((RHO_BLK, OUT), lambda i: (i, 0)),
        out_shape=jax.ShapeDtypeStruct((M, OUT), jnp.float32),
        compiler_params=pltpu.CompilerParams(
            dimension_semantics=("arbitrary",)),
    )(*partials, w1, b1.reshape(1, H), w2, b2.reshape(1, OUT))


@jax.jit
def kernel(f_subgraphs, phi_w1, phi_b1, phi_w2, phi_b2,
           rho_w1, rho_b1, rho_w2, rho_b2, segment_ids):
    ids = segment_ids.astype(jnp.int32)
    zeros = jnp.zeros((SEG_BLK, DH), jnp.float32)
    w1 = phi_w1.astype(jnp.bfloat16)
    w2 = phi_w2.astype(jnp.bfloat16)
    b1 = phi_b1.reshape(1, H)
    b2 = phi_b2.reshape(1, D)

    partials = []
    row0 = 0
    for count in STAGE_SIZES:
        ph = _phi_stage(f_subgraphs, w1, b1, w2, b2, row0, count)
        partials.append(_segment_sum_stage(ph, ids, zeros, row0, count))
        row0 += count
    return _rho(partials, rho_w1, rho_b1, rho_w2, rho_b2)
